# initial kernel scaffold (unmeasured)
import jax
import jax.numpy as jnp
from jax import lax
from jax.experimental import pallas as pl
from jax.experimental.pallas import tpu as pltpu


def kernel(x, pi):
    def body(x_ref, pi_ref, out_ref, send_sem, recv_sem):
        my_x = lax.axis_index("x")
        my_y = lax.axis_index("y")
        my_z = lax.axis_index("z")
        other = (1 - my_x, my_y, my_z)

        barrier = pltpu.get_barrier_semaphore()
        pl.semaphore_signal(
            barrier, inc=1, device_id=other, device_id_type=pl.DeviceIdType.MESH
        )
        pl.semaphore_wait(barrier, 1)

        swap = pi_ref[my_x] != my_x

        @pl.when(swap)
        def _():
            rdma = pltpu.make_async_remote_copy(
                src_ref=x_ref,
                dst_ref=out_ref,
                send_sem=send_sem,
                recv_sem=recv_sem,
                device_id=other,
                device_id_type=pl.DeviceIdType.MESH,
            )
            rdma.start()
            rdma.wait()

        @pl.when(jnp.logical_not(swap))
        def _():
            out_ref[...] = x_ref[...]

    return pl.pallas_call(
        body,
        out_shape=jax.ShapeDtypeStruct(x.shape, x.dtype),
        in_specs=[
            pl.BlockSpec(memory_space=pltpu.VMEM),
            pl.BlockSpec(memory_space=pltpu.SMEM),
        ],
        out_specs=pl.BlockSpec(memory_space=pltpu.VMEM),
        scratch_shapes=[
            pltpu.SemaphoreType.DMA,
            pltpu.SemaphoreType.DMA,
        ],
        compiler_params=pltpu.CompilerParams(collective_id=0),
    )(x, pi)


# baseline (device time: 387638 ns/iter reference)
import jax
import jax.numpy as jnp
from jax import lax
from jax.experimental import pallas as pl
from jax.experimental.pallas import tpu as pltpu


def kernel(x, pi):
    def body(x_ref, pi_ref, out_ref, send_sem, recv_sem):
        my_x = lax.axis_index("x")
        my_y = lax.axis_index("y")
        my_z = lax.axis_index("z")
        other = (1 - my_x, my_y, my_z)

        barrier = pltpu.get_barrier_semaphore()
        pl.semaphore_signal(
            barrier, inc=1, device_id=other, device_id_type=pl.DeviceIdType.MESH
        )
        pl.semaphore_wait(barrier, 1)

        swap = pi_ref[my_x] != my_x

        @pl.when(swap)
        def _():
            rdma = pltpu.make_async_remote_copy(
                src_ref=x_ref,
                dst_ref=out_ref,
                send_sem=send_sem,
                recv_sem=recv_sem,
                device_id=other,
                device_id_type=pl.DeviceIdType.MESH,
            )
            rdma.start()
            rdma.wait()

        @pl.when(jnp.logical_not(swap))
        def _():
            copy = pltpu.make_async_copy(x_ref, out_ref, send_sem)
            copy.start()
            copy.wait()

    return pl.pallas_call(
        body,
        out_shape=jax.ShapeDtypeStruct(x.shape, x.dtype),
        in_specs=[
            pl.BlockSpec(memory_space=pl.ANY),
            pl.BlockSpec(memory_space=pltpu.SMEM),
        ],
        out_specs=pl.BlockSpec(memory_space=pl.ANY),
        scratch_shapes=[
            pltpu.SemaphoreType.DMA,
            pltpu.SemaphoreType.DMA,
        ],
        compiler_params=pltpu.CompilerParams(collective_id=0),
    )(x, pi)


# device time: 192563 ns/iter; 2.0130x vs baseline; 2.0130x over previous
import jax
import jax.numpy as jnp
from jax import lax
from jax.experimental import pallas as pl
from jax.experimental.pallas import tpu as pltpu

N_CHUNK = 8
N_IN_SLOTS = 2


def kernel(x, pi):
    _, m, n = x.shape
    rows = m // N_CHUNK

    def body(x_ref, pi_ref, out_ref, xin, bfbuf, in_sems, send_sems, recv_sems):
        my_x = lax.axis_index("x")
        my_y = lax.axis_index("y")
        my_z = lax.axis_index("z")
        other = (1 - my_x, my_y, my_z)

        barrier = pltpu.get_barrier_semaphore()
        pl.semaphore_signal(
            barrier, inc=1, device_id=other, device_id_type=pl.DeviceIdType.MESH
        )
        pl.semaphore_wait(barrier, 1)

        swap = pi_ref[my_x] != my_x

        def load(k, slot):
            return pltpu.make_async_copy(
                x_ref.at[0, pl.ds(k * rows, rows), :],
                xin.at[slot],
                in_sems.at[slot],
            )

        def remote_send(k):
            return pltpu.make_async_remote_copy(
                src_ref=bfbuf.at[k],
                dst_ref=out_ref.at[0, pl.ds(k * rows, rows), :],
                send_sem=send_sems.at[k],
                recv_sem=recv_sems.at[k],
                device_id=other,
                device_id_type=pl.DeviceIdType.MESH,
            )

        def local_store(k):
            return pltpu.make_async_copy(
                bfbuf.at[k],
                out_ref.at[0, pl.ds(k * rows, rows), :],
                send_sems.at[k],
            )

        load(0, 0).start()
        for k in range(N_CHUNK):
            if k + 1 < N_CHUNK:
                load(k + 1, (k + 1) % N_IN_SLOTS).start()
            load(k, k % N_IN_SLOTS).wait()
            bfbuf[k] = xin[k % N_IN_SLOTS].astype(jnp.bfloat16)

            @pl.when(swap)
            def _():
                remote_send(k).start()

            @pl.when(jnp.logical_not(swap))
            def _():
                local_store(k).start()

        for k in range(N_CHUNK):

            @pl.when(swap)
            def _():
                remote_send(k).wait_send()
                remote_send(k).wait_recv()

            @pl.when(jnp.logical_not(swap))
            def _():
                local_store(k).wait()

    return pl.pallas_call(
        body,
        out_shape=jax.ShapeDtypeStruct(x.shape, jnp.bfloat16),
        in_specs=[
            pl.BlockSpec(memory_space=pl.ANY),
            pl.BlockSpec(memory_space=pltpu.SMEM),
        ],
        out_specs=pl.BlockSpec(memory_space=pl.ANY),
        scratch_shapes=[
            pltpu.VMEM((N_IN_SLOTS, rows, n), jnp.float32),
            pltpu.VMEM((N_CHUNK, rows, n), jnp.bfloat16),
            pltpu.SemaphoreType.DMA((N_IN_SLOTS,)),
            pltpu.SemaphoreType.DMA((N_CHUNK,)),
            pltpu.SemaphoreType.DMA((N_CHUNK,)),
        ],
        compiler_params=pltpu.CompilerParams(collective_id=0),
    )(x, pi)


# device time: 191935 ns/iter; 2.0196x vs baseline; 1.0033x over previous
import jax
import jax.numpy as jnp
from jax import lax
from jax.experimental import pallas as pl
from jax.experimental.pallas import tpu as pltpu

N_CHUNK = 16
N_IN_SLOTS = 3


def kernel(x, pi):
    _, m, n = x.shape
    rows = m // N_CHUNK

    def body(x_ref, pi_ref, out_ref, xin, bfbuf, in_sems, send_sems, recv_sems):
        my_x = lax.axis_index("x")
        my_y = lax.axis_index("y")
        my_z = lax.axis_index("z")
        other = (1 - my_x, my_y, my_z)

        swap = pi_ref[my_x] != my_x

        def load(k, slot):
            return pltpu.make_async_copy(
                x_ref.at[0, pl.ds(k * rows, rows), :],
                xin.at[slot],
                in_sems.at[slot],
            )

        def remote_send(k):
            return pltpu.make_async_remote_copy(
                src_ref=bfbuf.at[k],
                dst_ref=out_ref.at[0, pl.ds(k * rows, rows), :],
                send_sem=send_sems.at[k],
                recv_sem=recv_sems.at[k],
                device_id=other,
                device_id_type=pl.DeviceIdType.MESH,
            )

        def local_store(k):
            return pltpu.make_async_copy(
                bfbuf.at[k],
                out_ref.at[0, pl.ds(k * rows, rows), :],
                send_sems.at[k],
            )

        for k in range(min(N_IN_SLOTS, N_CHUNK)):
            load(k, k).start()

        barrier = pltpu.get_barrier_semaphore()
        pl.semaphore_signal(
            barrier, inc=1, device_id=other, device_id_type=pl.DeviceIdType.MESH
        )
        pl.semaphore_wait(barrier, 1)

        for k in range(N_CHUNK):
            load(k, k % N_IN_SLOTS).wait()
            bfbuf[k] = xin[k % N_IN_SLOTS].astype(jnp.bfloat16)
            if k + N_IN_SLOTS < N_CHUNK:
                load(k + N_IN_SLOTS, k % N_IN_SLOTS).start()

            @pl.when(swap)
            def _():
                remote_send(k).start()

            @pl.when(jnp.logical_not(swap))
            def _():
                local_store(k).start()

        for k in range(N_CHUNK):

            @pl.when(swap)
            def _():
                remote_send(k).wait_send()
                remote_send(k).wait_recv()

            @pl.when(jnp.logical_not(swap))
            def _():
                local_store(k).wait()

    return pl.pallas_call(
        body,
        out_shape=jax.ShapeDtypeStruct(x.shape, jnp.bfloat16),
        in_specs=[
            pl.BlockSpec(memory_space=pl.ANY),
            pl.BlockSpec(memory_space=pltpu.SMEM),
        ],
        out_specs=pl.BlockSpec(memory_space=pl.ANY),
        scratch_shapes=[
            pltpu.VMEM((N_IN_SLOTS, rows, n), jnp.float32),
            pltpu.VMEM((N_CHUNK, rows, n), jnp.bfloat16),
            pltpu.SemaphoreType.DMA((N_IN_SLOTS,)),
            pltpu.SemaphoreType.DMA((N_CHUNK,)),
            pltpu.SemaphoreType.DMA((N_CHUNK,)),
        ],
        compiler_params=pltpu.CompilerParams(collective_id=0),
    )(x, pi)


# device time: 93464 ns/iter; 4.1475x vs baseline; 2.0536x over previous
import jax
import jax.numpy as jnp
from jax import lax
from jax.experimental import pallas as pl
from jax.experimental.pallas import tpu as pltpu

CHUNK = 256
CLS = 1024
NXC = 4
NYC = 6
NZC = 6


def kernel(x, pi):
    _, m, n = x.shape
    n_tot = m // CHUNK

    def body(
        x_ref,
        pi_ref,
        out_ref,
        loadbuf,
        sendbuf,
        rbx,
        rby,
        rbz,
        ld_sems,
        sx,
        rx,
        sy,
        ry,
        sz,
        rz,
        st,
    ):
        mx = lax.axis_index("x")
        my = lax.axis_index("y")
        mz = lax.axis_index("z")
        zbit = mz % 2
        xn = (1 - mx, my, mz)
        yn = (mx, 1 - my, mz)
        zn = (mx, my, mz + 1 - 2 * zbit)

        c_me = 2 * my + zbit
        c_yn = 2 * (1 - my) + zbit
        c_zn = 2 * my + (1 - zbit)
        c_dg = 2 * (1 - my) + (1 - zbit)

        swap = pi_ref[mx] != mx

        def load(row_start, slot):
            return pltpu.make_async_copy(
                x_ref.at[0, pl.ds(row_start, CHUNK), :],
                loadbuf.at[slot],
                ld_sems.at[slot],
            )

        def rdma(src, dst, ssem, rsem, dev):
            return pltpu.make_async_remote_copy(
                src_ref=src,
                dst_ref=dst,
                send_sem=ssem,
                recv_sem=rsem,
                device_id=dev,
                device_id_type=pl.DeviceIdType.MESH,
            )

        def store(src, row_start, sem):
            return pltpu.make_async_copy(
                src, out_ref.at[0, pl.ds(row_start, CHUNK), :], sem
            )

        def x_flow(i):
            return rdma(sendbuf.at[i], rbx.at[i], sx.at[i], rx.at[i], xn)

        def y_flow(i):
            src = rbx.at[i] if i < NXC else rbz.at[i - NXC]
            return rdma(src, rby.at[i], sy.at[i], ry.at[i], yn)

        def z_flow(i):
            src = rbx.at[i] if i < NXC else rby.at[i - 2]
            return rdma(src, rbz.at[i], sz.at[i], rz.at[i], zn)

        for i in range(NXC):

            @pl.when(swap)
            def _(i=i):
                load(c_me * CLS + i * CHUNK, i).start()

            @pl.when(jnp.logical_not(swap))
            def _(i=i):
                load(i * CHUNK, i).start()

        barrier = pltpu.get_barrier_semaphore()
        for nbr in (xn, yn, zn):
            pl.semaphore_signal(
                barrier, inc=1, device_id=nbr, device_id_type=pl.DeviceIdType.MESH
            )
        pl.semaphore_wait(barrier, 3)

        @pl.when(swap)
        def _():
            stores = []

            def start_store(src, row_start, sem_idx):
                stores.append((src, row_start, sem_idx))
                store(src, row_start, st.at[sem_idx]).start()

            for i in range(NXC):
                load(c_me * CLS + i * CHUNK, i).wait()
                sendbuf[i] = loadbuf[i].astype(jnp.bfloat16)
                x_flow(i).start()

            for i in range(NXC):
                x_flow(i).wait_recv()
                y_flow(i).start()
                z_flow(i).start()
                start_store(rbx.at[i], c_me * CLS + i * CHUNK, i)

            for i in range(2):
                z_flow(i).wait_recv()
                y_flow(NXC + i).start()
                start_store(rbz.at[i], c_zn * CLS + i * CHUNK, 4 + i)

            for i in (2, 3):
                y_flow(i).wait_recv()
                z_flow(NXC + i - 2).start()
                start_store(rby.at[i], c_yn * CLS + i * CHUNK, 6 + i - 2)

            for k, i in enumerate((0, 1)):
                y_flow(i).wait_recv()
                start_store(rby.at[i], c_yn * CLS + i * CHUNK, 8 + k)
            for k, i in enumerate((2, 3)):
                z_flow(i).wait_recv()
                start_store(rbz.at[i], c_zn * CLS + i * CHUNK, 10 + k)
            for k, i in enumerate((4, 5)):
                y_flow(i).wait_recv()
                start_store(rby.at[i], c_dg * CLS + (i - 4) * CHUNK, 12 + k)
            for k, i in enumerate((4, 5)):
                z_flow(i).wait_recv()
                start_store(rbz.at[i], c_dg * CLS + 512 + (i - 4) * CHUNK, 14 + k)

            for i in range(NXC):
                x_flow(i).wait_send()
            for i in range(NYC):
                y_flow(i).wait_send()
            for i in range(NZC):
                z_flow(i).wait_send()
            for src, row_start, sem_idx in stores:
                store(src, row_start, st.at[sem_idx]).wait()

        @pl.when(jnp.logical_not(swap))
        def _():
            for k in range(n_tot):
                load(k * CHUNK, k % NXC).wait()
                if k >= NXC:
                    store(
                        sendbuf.at[(k - NXC) % NXC],
                        (k - NXC) * CHUNK,
                        st.at[k - NXC],
                    ).wait()
                sendbuf[k % NXC] = loadbuf[k % NXC].astype(jnp.bfloat16)
                store(sendbuf.at[k % NXC], k * CHUNK, st.at[k]).start()
                if k + NXC < n_tot:
                    load((k + NXC) * CHUNK, k % NXC).start()
            for k in range(n_tot - NXC, n_tot):
                store(sendbuf.at[k % NXC], k * CHUNK, st.at[k]).wait()

    return pl.pallas_call(
        body,
        out_shape=jax.ShapeDtypeStruct(x.shape, jnp.bfloat16),
        in_specs=[
            pl.BlockSpec(memory_space=pl.ANY),
            pl.BlockSpec(memory_space=pltpu.SMEM),
        ],
        out_specs=pl.BlockSpec(memory_space=pl.ANY),
        scratch_shapes=[
            pltpu.VMEM((NXC, CHUNK, n), jnp.float32),
            pltpu.VMEM((NXC, CHUNK, n), jnp.bfloat16),
            pltpu.VMEM((NXC, CHUNK, n), jnp.bfloat16),
            pltpu.VMEM((NYC, CHUNK, n), jnp.bfloat16),
            pltpu.VMEM((NZC, CHUNK, n), jnp.bfloat16),
            pltpu.SemaphoreType.DMA((NXC,)),
            pltpu.SemaphoreType.DMA((NXC,)),
            pltpu.SemaphoreType.DMA((NXC,)),
            pltpu.SemaphoreType.DMA((NYC,)),
            pltpu.SemaphoreType.DMA((NYC,)),
            pltpu.SemaphoreType.DMA((NZC,)),
            pltpu.SemaphoreType.DMA((NZC,)),
            pltpu.SemaphoreType.DMA((16,)),
        ],
        compiler_params=pltpu.CompilerParams(collective_id=0),
    )(x, pi)


# device time: 88024 ns/iter; 4.4038x vs baseline; 1.0618x over previous
import jax
import jax.numpy as jnp
from jax import lax
from jax.experimental import pallas as pl
from jax.experimental.pallas import tpu as pltpu

CHUNK = 128
CLS = 1024
NXC = CLS // CHUNK
NH = 512 // CHUNK
NYC = NXC + NH
NZC = NXC + NH


def kernel(x, pi):
    _, m, n = x.shape
    n_tot = m // CHUNK

    def body(
        x_ref,
        pi_ref,
        out_ref,
        loadbuf,
        sendbuf,
        rbx,
        rby,
        rbz,
        ld_sems,
        sx,
        rx,
        sy,
        ry,
        sz,
        rz,
        st,
    ):
        mx = lax.axis_index("x")
        my = lax.axis_index("y")
        mz = lax.axis_index("z")
        zbit = mz % 2
        xn = (1 - mx, my, mz)
        yn = (mx, 1 - my, mz)
        zn = (mx, my, mz + 1 - 2 * zbit)

        c_me = 2 * my + zbit
        c_yn = 2 * (1 - my) + zbit
        c_zn = 2 * my + (1 - zbit)
        c_dg = 2 * (1 - my) + (1 - zbit)

        swap = pi_ref[mx] != mx

        def load(row_start, slot):
            return pltpu.make_async_copy(
                x_ref.at[0, pl.ds(row_start, CHUNK), :],
                loadbuf.at[slot],
                ld_sems.at[slot],
            )

        def rdma(src, dst, ssem, rsem, dev):
            return pltpu.make_async_remote_copy(
                src_ref=src,
                dst_ref=dst,
                send_sem=ssem,
                recv_sem=rsem,
                device_id=dev,
                device_id_type=pl.DeviceIdType.MESH,
            )

        def store(src, row_start, sem):
            return pltpu.make_async_copy(
                src, out_ref.at[0, pl.ds(row_start, CHUNK), :], sem
            )

        def x_flow(i):
            return rdma(sendbuf.at[i], rbx.at[i], sx.at[i], rx.at[i], xn)

        def y_flow(i):
            src = rbx.at[i] if i < NXC else rbz.at[i - NXC]
            return rdma(src, rby.at[i], sy.at[i], ry.at[i], yn)

        def z_flow(i):
            src = rbx.at[i] if i < NXC else rby.at[NH + i - NXC]
            return rdma(src, rbz.at[i], sz.at[i], rz.at[i], zn)

        for i in range(NXC):

            @pl.when(swap)
            def _(i=i):
                load(c_me * CLS + i * CHUNK, i).start()

            @pl.when(jnp.logical_not(swap))
            def _(i=i):
                load(i * CHUNK, i).start()

        barrier = pltpu.get_barrier_semaphore()
        for nbr in (xn, yn, zn):
            pl.semaphore_signal(
                barrier, inc=1, device_id=nbr, device_id_type=pl.DeviceIdType.MESH
            )
        pl.semaphore_wait(barrier, 3)

        @pl.when(swap)
        def _():
            stores = []

            def start_store(src, row_start):
                sem_idx = len(stores)
                stores.append((src, row_start, sem_idx))
                store(src, row_start, st.at[sem_idx]).start()

            for i in range(NXC):
                load(c_me * CLS + i * CHUNK, i).wait()
                sendbuf[i] = loadbuf[i].astype(jnp.bfloat16)
                x_flow(i).start()

            for i in range(NXC):
                x_flow(i).wait_recv()
                y_flow(i).start()
                z_flow(i).start()
                start_store(rbx.at[i], c_me * CLS + i * CHUNK)

            for i in range(NH):
                z_flow(i).wait_recv()
                y_flow(NXC + i).start()
                start_store(rbz.at[i], c_zn * CLS + i * CHUNK)

            for i in range(NH, NXC):
                y_flow(i).wait_recv()
                z_flow(NXC + i - NH).start()
                start_store(rby.at[i], c_yn * CLS + i * CHUNK)

            for i in range(NH):
                y_flow(i).wait_recv()
                start_store(rby.at[i], c_yn * CLS + i * CHUNK)
            for i in range(NH, NXC):
                z_flow(i).wait_recv()
                start_store(rbz.at[i], c_zn * CLS + i * CHUNK)
            for i in range(NXC, NYC):
                y_flow(i).wait_recv()
                start_store(rby.at[i], c_dg * CLS + (i - NXC) * CHUNK)
            for i in range(NXC, NZC):
                z_flow(i).wait_recv()
                start_store(rbz.at[i], c_dg * CLS + 512 + (i - NXC) * CHUNK)

            for i in range(NXC):
                x_flow(i).wait_send()
            for i in range(NYC):
                y_flow(i).wait_send()
            for i in range(NZC):
                z_flow(i).wait_send()
            for src, row_start, sem_idx in stores:
                store(src, row_start, st.at[sem_idx]).wait()

        @pl.when(jnp.logical_not(swap))
        def _():
            for k in range(n_tot):
                load(k * CHUNK, k % NXC).wait()
                if k >= NXC:
                    store(
                        sendbuf.at[(k - NXC) % NXC],
                        (k - NXC) * CHUNK,
                        st.at[k - NXC],
                    ).wait()
                sendbuf[k % NXC] = loadbuf[k % NXC].astype(jnp.bfloat16)
                store(sendbuf.at[k % NXC], k * CHUNK, st.at[k]).start()
                if k + NXC < n_tot:
                    load((k + NXC) * CHUNK, k % NXC).start()
            for k in range(n_tot - NXC, n_tot):
                store(sendbuf.at[k % NXC], k * CHUNK, st.at[k]).wait()

    return pl.pallas_call(
        body,
        out_shape=jax.ShapeDtypeStruct(x.shape, jnp.bfloat16),
        in_specs=[
            pl.BlockSpec(memory_space=pl.ANY),
            pl.BlockSpec(memory_space=pltpu.SMEM),
        ],
        out_specs=pl.BlockSpec(memory_space=pl.ANY),
        scratch_shapes=[
            pltpu.VMEM((NXC, CHUNK, n), jnp.float32),
            pltpu.VMEM((NXC, CHUNK, n), jnp.bfloat16),
            pltpu.VMEM((NXC, CHUNK, n), jnp.bfloat16),
            pltpu.VMEM((NYC, CHUNK, n), jnp.bfloat16),
            pltpu.VMEM((NZC, CHUNK, n), jnp.bfloat16),
            pltpu.SemaphoreType.DMA((NXC,)),
            pltpu.SemaphoreType.DMA((NXC,)),
            pltpu.SemaphoreType.DMA((NXC,)),
            pltpu.SemaphoreType.DMA((NYC,)),
            pltpu.SemaphoreType.DMA((NYC,)),
            pltpu.SemaphoreType.DMA((NZC,)),
            pltpu.SemaphoreType.DMA((NZC,)),
            pltpu.SemaphoreType.DMA((n_tot,)),
        ],
        compiler_params=pltpu.CompilerParams(collective_id=0),
    )(x, pi)


# device time: 82431 ns/iter; 4.7026x vs baseline; 1.0679x over previous
import jax
import jax.numpy as jnp
from jax import lax
from jax.experimental import pallas as pl
from jax.experimental.pallas import tpu as pltpu

CHUNK = 128
CLS = 896
NCC = CLS // CHUNK
XSTART = 4 * CLS
NXE = 4
NXF = NCC + NXE
NHA = 4
NHB = 3
NYF = NCC + NHA
NZF = NCC + NHB


def kernel(x, pi):
    _, m, n = x.shape
    n_tot = m // CHUNK

    def body(
        x_ref,
        pi_ref,
        out_ref,
        loadbuf,
        sendbuf,
        rbx,
        rby,
        rbz,
        ld_sems,
        sx,
        rx,
        sy,
        ry,
        sz,
        rz,
        st,
    ):
        mx = lax.axis_index("x")
        my = lax.axis_index("y")
        mz = lax.axis_index("z")
        zbit = mz % 2
        xn = (1 - mx, my, mz)
        yn = (mx, 1 - my, mz)
        zn = (mx, my, mz + 1 - 2 * zbit)

        c_me = 2 * my + zbit
        c_yn = 2 * (1 - my) + zbit
        c_zn = 2 * my + (1 - zbit)
        c_dg = 2 * (1 - my) + (1 - zbit)

        swap = pi_ref[mx] != mx

        def load(row_start, slot):
            return pltpu.make_async_copy(
                x_ref.at[0, pl.ds(row_start, CHUNK), :],
                loadbuf.at[slot],
                ld_sems.at[slot],
            )

        def rdma(src, dst, ssem, rsem, dev):
            return pltpu.make_async_remote_copy(
                src_ref=src,
                dst_ref=dst,
                send_sem=ssem,
                recv_sem=rsem,
                device_id=dev,
                device_id_type=pl.DeviceIdType.MESH,
            )

        def store(src, row_start, sem):
            return pltpu.make_async_copy(
                src, out_ref.at[0, pl.ds(row_start, CHUNK), :], sem
            )

        def send_row(i):
            if i < NCC:
                return c_me * CLS + i * CHUNK
            return XSTART + (i - NCC) * CHUNK

        def x_flow(i):
            if i < NCC:
                dst = rbx.at[i]
            else:
                dst = out_ref.at[0, pl.ds(XSTART + (i - NCC) * CHUNK, CHUNK), :]
            return rdma(sendbuf.at[i], dst, sx.at[i], rx.at[i], xn)

        def y_flow(i):
            src = rbx.at[i] if i < NCC else rbz.at[i - NCC]
            return rdma(src, rby.at[i], sy.at[i], ry.at[i], yn)

        def z_flow(i):
            src = rbx.at[i] if i < NCC else rby.at[NHA + i - NCC]
            return rdma(src, rbz.at[i], sz.at[i], rz.at[i], zn)

        for i in range(NXF):

            @pl.when(swap)
            def _(i=i):
                load(send_row(i), i).start()

            @pl.when(jnp.logical_not(swap))
            def _(i=i):
                load(i * CHUNK, i).start()

        barrier = pltpu.get_barrier_semaphore()
        for nbr in (xn, yn, zn):
            pl.semaphore_signal(
                barrier, inc=1, device_id=nbr, device_id_type=pl.DeviceIdType.MESH
            )
        pl.semaphore_wait(barrier, 3)

        @pl.when(swap)
        def _():
            stores = []

            def start_store(src, row_start):
                sem_idx = len(stores)
                stores.append((src, row_start, sem_idx))
                store(src, row_start, st.at[sem_idx]).start()

            for i in range(NXF):
                load(send_row(i), i).wait()
                sendbuf[i] = loadbuf[i].astype(jnp.bfloat16)
                x_flow(i).start()

            for i in range(NCC):
                x_flow(i).wait_recv()
                y_flow(i).start()
                z_flow(i).start()
                start_store(rbx.at[i], c_me * CLS + i * CHUNK)

            for i in range(NHA):
                z_flow(i).wait_recv()
                y_flow(NCC + i).start()
                start_store(rbz.at[i], c_zn * CLS + i * CHUNK)

            for i in range(NHA, NCC):
                y_flow(i).wait_recv()
                z_flow(NCC + i - NHA).start()
                start_store(rby.at[i], c_yn * CLS + i * CHUNK)

            for i in range(NHA):
                y_flow(i).wait_recv()
                start_store(rby.at[i], c_yn * CLS + i * CHUNK)
            for i in range(NHA, NCC):
                z_flow(i).wait_recv()
                start_store(rbz.at[i], c_zn * CLS + i * CHUNK)
            for i in range(NCC, NYF):
                y_flow(i).wait_recv()
                start_store(rby.at[i], c_dg * CLS + (i - NCC) * CHUNK)
            for i in range(NCC, NZF):
                z_flow(i).wait_recv()
                start_store(rbz.at[i], c_dg * CLS + 512 + (i - NCC) * CHUNK)
            for i in range(NCC, NXF):
                x_flow(i).wait_recv()

            for i in range(NXF):
                x_flow(i).wait_send()
            for i in range(NYF):
                y_flow(i).wait_send()
            for i in range(NZF):
                z_flow(i).wait_send()
            for src, row_start, sem_idx in stores:
                store(src, row_start, st.at[sem_idx]).wait()

        @pl.when(jnp.logical_not(swap))
        def _():
            for k in range(n_tot):
                load(k * CHUNK, k % NXF).wait()
                if k >= NXF:
                    store(
                        sendbuf.at[(k - NXF) % NXF],
                        (k - NXF) * CHUNK,
                        st.at[k - NXF],
                    ).wait()
                sendbuf[k % NXF] = loadbuf[k % NXF].astype(jnp.bfloat16)
                store(sendbuf.at[k % NXF], k * CHUNK, st.at[k]).start()
                if k + NXF < n_tot:
                    load((k + NXF) * CHUNK, k % NXF).start()
            for k in range(n_tot - NXF, n_tot):
                store(sendbuf.at[k % NXF], k * CHUNK, st.at[k]).wait()

    return pl.pallas_call(
        body,
        out_shape=jax.ShapeDtypeStruct(x.shape, jnp.bfloat16),
        in_specs=[
            pl.BlockSpec(memory_space=pl.ANY),
            pl.BlockSpec(memory_space=pltpu.SMEM),
        ],
        out_specs=pl.BlockSpec(memory_space=pl.ANY),
        scratch_shapes=[
            pltpu.VMEM((NXF, CHUNK, n), jnp.float32),
            pltpu.VMEM((NXF, CHUNK, n), jnp.bfloat16),
            pltpu.VMEM((NCC, CHUNK, n), jnp.bfloat16),
            pltpu.VMEM((NYF, CHUNK, n), jnp.bfloat16),
            pltpu.VMEM((NZF, CHUNK, n), jnp.bfloat16),
            pltpu.SemaphoreType.DMA((NXF,)),
            pltpu.SemaphoreType.DMA((NXF,)),
            pltpu.SemaphoreType.DMA((NXF,)),
            pltpu.SemaphoreType.DMA((NYF,)),
            pltpu.SemaphoreType.DMA((NYF,)),
            pltpu.SemaphoreType.DMA((NZF,)),
            pltpu.SemaphoreType.DMA((NZF,)),
            pltpu.SemaphoreType.DMA((n_tot,)),
        ],
        compiler_params=pltpu.CompilerParams(collective_id=0),
    )(x, pi)


# device time: 78072 ns/iter; 4.9651x vs baseline; 1.0558x over previous
import jax
import jax.numpy as jnp
from jax import lax
from jax.experimental import pallas as pl
from jax.experimental.pallas import tpu as pltpu

CHUNK = 64
CLS = 896
NCC = CLS // CHUNK
XSTART = 4 * CLS
NXE = 512 // CHUNK
NXF = NCC + NXE
NHA = 7
NHB = 7
NYF = NCC + NHA
NZF = NCC + NHB


def kernel(x, pi):
    _, m, n = x.shape
    n_tot = m // CHUNK

    def body(
        x_ref,
        pi_ref,
        out_ref,
        loadbuf,
        sendbuf,
        rbx,
        rby,
        rbz,
        ld_sems,
        sx,
        rx,
        sy,
        ry,
        sz,
        rz,
        st,
    ):
        mx = lax.axis_index("x")
        my = lax.axis_index("y")
        mz = lax.axis_index("z")
        zbit = mz % 2
        xn = (1 - mx, my, mz)
        yn = (mx, 1 - my, mz)
        zn = (mx, my, mz + 1 - 2 * zbit)

        c_me = 2 * my + zbit
        c_yn = 2 * (1 - my) + zbit
        c_zn = 2 * my + (1 - zbit)
        c_dg = 2 * (1 - my) + (1 - zbit)

        swap = pi_ref[mx] != mx

        def load(row_start, slot):
            return pltpu.make_async_copy(
                x_ref.at[0, pl.ds(row_start, CHUNK), :],
                loadbuf.at[slot],
                ld_sems.at[slot],
            )

        def rdma(src, dst, ssem, rsem, dev):
            return pltpu.make_async_remote_copy(
                src_ref=src,
                dst_ref=dst,
                send_sem=ssem,
                recv_sem=rsem,
                device_id=dev,
                device_id_type=pl.DeviceIdType.MESH,
            )

        def store(src, row_start, sem):
            return pltpu.make_async_copy(
                src, out_ref.at[0, pl.ds(row_start, CHUNK), :], sem
            )

        def send_row(i):
            if i < NCC:
                return c_me * CLS + i * CHUNK
            return XSTART + (i - NCC) * CHUNK

        def x_flow(i):
            if i < NCC:
                dst = rbx.at[i]
            else:
                dst = out_ref.at[0, pl.ds(XSTART + (i - NCC) * CHUNK, CHUNK), :]
            return rdma(sendbuf.at[i], dst, sx.at[i], rx.at[i], xn)

        def y_flow(i):
            src = rbx.at[i] if i < NCC else rbz.at[i - NCC]
            return rdma(src, rby.at[i], sy.at[i], ry.at[i], yn)

        def z_flow(i):
            src = rbx.at[i] if i < NCC else rby.at[NHA + i - NCC]
            return rdma(src, rbz.at[i], sz.at[i], rz.at[i], zn)

        for i in range(NXF):

            @pl.when(swap)
            def _(i=i):
                load(send_row(i), i).start()

            @pl.when(jnp.logical_not(swap))
            def _(i=i):
                load(i * CHUNK, i).start()

        barrier = pltpu.get_barrier_semaphore()
        for nbr in (xn, yn, zn):
            pl.semaphore_signal(
                barrier, inc=1, device_id=nbr, device_id_type=pl.DeviceIdType.MESH
            )
        pl.semaphore_wait(barrier, 3)

        @pl.when(swap)
        def _():
            stores = []

            def start_store(src, row_start):
                sem_idx = len(stores)
                stores.append((src, row_start, sem_idx))
                store(src, row_start, st.at[sem_idx]).start()

            for i in range(NXF):
                load(send_row(i), i).wait()
                sendbuf[i] = loadbuf[i].astype(jnp.bfloat16)
                x_flow(i).start()

            for i in range(NCC):
                x_flow(i).wait_recv()
                y_flow(i).start()
                z_flow(i).start()
                start_store(rbx.at[i], c_me * CLS + i * CHUNK)

            for i in range(NHA):
                z_flow(i).wait_recv()
                y_flow(NCC + i).start()
                start_store(rbz.at[i], c_zn * CLS + i * CHUNK)

            for i in range(NHA, NCC):
                y_flow(i).wait_recv()
                z_flow(NCC + i - NHA).start()
                start_store(rby.at[i], c_yn * CLS + i * CHUNK)

            for i in range(NHA):
                y_flow(i).wait_recv()
                start_store(rby.at[i], c_yn * CLS + i * CHUNK)
            for i in range(NHA, NCC):
                z_flow(i).wait_recv()
                start_store(rbz.at[i], c_zn * CLS + i * CHUNK)
            for i in range(NCC, NYF):
                y_flow(i).wait_recv()
                start_store(rby.at[i], c_dg * CLS + (i - NCC) * CHUNK)
            for i in range(NCC, NZF):
                z_flow(i).wait_recv()
                start_store(rbz.at[i], c_dg * CLS + NHA * CHUNK + (i - NCC) * CHUNK)
            for i in range(NCC, NXF):
                x_flow(i).wait_recv()

            for i in range(NXF):
                x_flow(i).wait_send()
            for i in range(NYF):
                y_flow(i).wait_send()
            for i in range(NZF):
                z_flow(i).wait_send()
            for src, row_start, sem_idx in stores:
                store(src, row_start, st.at[sem_idx]).wait()

        @pl.when(jnp.logical_not(swap))
        def _():
            for k in range(n_tot):
                load(k * CHUNK, k % NXF).wait()
                if k >= NXF:
                    store(
                        sendbuf.at[(k - NXF) % NXF],
                        (k - NXF) * CHUNK,
                        st.at[k - NXF],
                    ).wait()
                sendbuf[k % NXF] = loadbuf[k % NXF].astype(jnp.bfloat16)
                store(sendbuf.at[k % NXF], k * CHUNK, st.at[k]).start()
                if k + NXF < n_tot:
                    load((k + NXF) * CHUNK, k % NXF).start()
            for k in range(n_tot - NXF, n_tot):
                store(sendbuf.at[k % NXF], k * CHUNK, st.at[k]).wait()

    return pl.pallas_call(
        body,
        out_shape=jax.ShapeDtypeStruct(x.shape, jnp.bfloat16),
        in_specs=[
            pl.BlockSpec(memory_space=pl.ANY),
            pl.BlockSpec(memory_space=pltpu.SMEM),
        ],
        out_specs=pl.BlockSpec(memory_space=pl.ANY),
        scratch_shapes=[
            pltpu.VMEM((NXF, CHUNK, n), jnp.float32),
            pltpu.VMEM((NXF, CHUNK, n), jnp.bfloat16),
            pltpu.VMEM((NCC, CHUNK, n), jnp.bfloat16),
            pltpu.VMEM((NYF, CHUNK, n), jnp.bfloat16),
            pltpu.VMEM((NZF, CHUNK, n), jnp.bfloat16),
            pltpu.SemaphoreType.DMA((NXF,)),
            pltpu.SemaphoreType.DMA((NXF,)),
            pltpu.SemaphoreType.DMA((NXF,)),
            pltpu.SemaphoreType.DMA((NYF,)),
            pltpu.SemaphoreType.DMA((NYF,)),
            pltpu.SemaphoreType.DMA((NZF,)),
            pltpu.SemaphoreType.DMA((NZF,)),
            pltpu.SemaphoreType.DMA((n_tot,)),
        ],
        compiler_params=pltpu.CompilerParams(collective_id=0),
    )(x, pi)
